# baseline (speedup 1.0000x reference)
import functools

import jax
import jax.numpy as jnp
from jax import lax
from jax.experimental import pallas as pl
from jax.experimental.pallas import tpu as pltpu
from jax.experimental.pallas import tpu_sc as plsc

N = 10000
D = 128
E = 320000
G = 512
T = 100

NC = 2
NS = 16
NW = NC * NS
C = 80
EPT = E // NW
NCHUNK = EPT // C
NGRP = 5
GRP = NCHUNK // NGRP
NBLK = N // C
NDEG = 10240
DEG_PT = NDEG // NS

_mesh = plsc.VectorSubcoreMesh(core_axis_name="c", subcore_axis_name="s")


def _zero_vmem_1d(ref, n):
    z16 = jnp.zeros((16,), jnp.float32)

    def body(i, carry):
        ref[pl.ds(i * 16, 16)] = z16
        return carry

    lax.fori_loop(0, n // 16, body, 0)


def _zero_vmem_2d(ref, rows, cols):
    z16 = jnp.zeros((16,), jnp.float32)

    def body(i, carry):
        r = i // (cols // 16)
        c = lax.rem(i, cols // 16) * 16
        ref[r, pl.ds(c, 16)] = z16
        return carry

    lax.fori_loop(0, rows * (cols // 16), body, 0)


@functools.partial(
    pl.kernel,
    out_type=jax.ShapeDtypeStruct((NC, NDEG), jnp.float32),
    mesh=_mesh,
    scratch_types=[
        pltpu.VMEM((NCHUNK, C), jnp.int32),
        pltpu.VMEM((C,), jnp.float32),
        pltpu.VMEM((DEG_PT,), jnp.float32),
        pltpu.VMEM_SHARED((NDEG,), jnp.float32),
    ],
)
def _deg_kernel(dst_hbm, degp_hbm, idx_v, ones_v, zbuf, deg_sh):
    cid = lax.axis_index("c")
    sid = lax.axis_index("s")
    w = cid * NS + sid

    pltpu.sync_copy(dst_hbm.at[w], idx_v)

    one16 = jnp.ones((16,), jnp.float32)
    for i in range(C // 16):
        ones_v[pl.ds(i * 16, 16)] = one16
    _zero_vmem_1d(zbuf, DEG_PT)
    pltpu.sync_copy(zbuf, deg_sh.at[pl.ds(sid * DEG_PT, DEG_PT)])
    plsc.subcore_barrier()

    def step(c, carry):
        pltpu.sync_copy(ones_v, deg_sh.at[idx_v.at[c]], add=True)
        return carry

    lax.fori_loop(0, NCHUNK, step, 0)
    plsc.subcore_barrier()

    pltpu.sync_copy(deg_sh.at[pl.ds(sid * DEG_PT, DEG_PT)],
                    degp_hbm.at[cid, pl.ds(sid * DEG_PT, DEG_PT)])


DEPTH = 4
HALF = N // NC
TRASH = HALF
NCH = 128
TARGET = 68
AROWS = 5056
NBLK2 = (HALF + C - 1) // C


@functools.partial(
    pl.kernel,
    out_type=(
        jax.ShapeDtypeStruct((2, NW, NCH, C), jnp.int32),
        jax.ShapeDtypeStruct((2, NW, NCH, C), jnp.int32),
        jax.ShapeDtypeStruct((NW * 16,), jnp.int32),
    ),
    mesh=_mesh,
    compiler_params=pltpu.CompilerParams(needs_layout_passes=False),
    scratch_types=[
        pltpu.VMEM((EPT,), jnp.int32),
        pltpu.VMEM((EPT,), jnp.int32),
        pltpu.VMEM((NCH, C), jnp.int32),
        pltpu.VMEM((NCH, C), jnp.int32),
        pltpu.VMEM((NCH, C), jnp.int32),
        pltpu.VMEM((NCH, C), jnp.int32),
        pltpu.VMEM((16,), jnp.int32),
    ],
)
def _part_kernel(src_hbm, dst_hbm, srcp_hbm, dstp_hbm, cnt_hbm,
                 sin, din, src_a, dst_a, src_b, dst_b, cnts):
    cid = lax.axis_index("c")
    sid = lax.axis_index("s")
    p = cid * NS + sid
    pltpu.sync_copy(src_hbm.at[pl.ds(p * EPT, EPT)], sin)
    pltpu.sync_copy(dst_hbm.at[pl.ds(p * EPT, EPT)], din)

    lane = lax.broadcasted_iota(jnp.int32, (16,), 0)

    def rowcol(pos, m):
        r = jnp.where(m, pos // C, NCH - 1)
        c = jnp.where(m, pos % C, lane)
        return r, c

    def step(i, cnt):
        ca, cb = cnt
        s16 = sin[pl.ds(i * 16, 16)]
        d16 = din[pl.ds(i * 16, 16)]
        m_a = d16 < HALF
        m_b = jnp.logical_not(m_a)
        cs_a = plsc.cumsum(m_a.astype(jnp.int32))
        na = cs_a[15]
        ra, cla = rowcol(ca + cs_a - 1, m_a)
        rb, clb = rowcol(cb + (lane - cs_a), m_b)
        plsc.store_scatter(src_a, [ra, cla], s16)
        plsc.store_scatter(dst_a, [ra, cla], d16)
        plsc.store_scatter(src_b, [rb, clb], s16)
        plsc.store_scatter(dst_b, [rb, clb], d16 - HALF)
        return (ca + na, cb + (16 - na))

    ca, cb = lax.fori_loop(0, EPT // 16, step,
                           (jnp.int32(0), jnp.int32(0)))

    zsrc = jnp.zeros((16,), jnp.int32)
    tdst = jnp.full((16,), TRASH, jnp.int32)
    tmask = jnp.ones((16,), jnp.bool_)

    def pad(ref_s, ref_d, cnt):
        nch = (cnt + C - 1) // C
        nch = jnp.maximum((nch + DEPTH - 1) // DEPTH * DEPTH, TARGET)
        end = nch * C

        def fill(k, carry):
            off = cnt + k * 16

            @pl.when(off < end)
            def _():
                offs = jnp.minimum(off + lane, NCH * C - 1)
                r = offs // C
                c = offs % C
                plsc.store_scatter(ref_s, [r, c], zsrc)
                plsc.store_scatter(ref_d, [r, c], tdst)
            return carry

        lax.fori_loop(0, TARGET * C // 16 + 1, fill, 0)
        return nch

    nch_a = pad(src_a, dst_a, ca)
    nch_b = pad(src_b, dst_b, cb)

    cnts[...] = jnp.where(lane == 0, nch_a, jnp.where(lane == 1, nch_b, 0))

    pltpu.sync_copy(src_a, srcp_hbm.at[0, p])
    pltpu.sync_copy(dst_a, dstp_hbm.at[0, p])
    pltpu.sync_copy(src_b, srcp_hbm.at[1, p])
    pltpu.sync_copy(dst_b, dstp_hbm.at[1, p])
    pltpu.sync_copy(cnts, cnt_hbm.at[pl.ds(p * 16, 16)])


@functools.partial(
    pl.kernel,
    out_type=jax.ShapeDtypeStruct((NC, AROWS, D), jnp.float32),
    mesh=_mesh,
    scratch_types=[
        pltpu.VMEM((NCH, C), jnp.int32),
        pltpu.VMEM((NCH, C), jnp.int32),
        pltpu.VMEM((16,), jnp.int32),
        pltpu.VMEM((DEPTH, C, D), jnp.float32),
        pltpu.VMEM_SHARED((AROWS, D), jnp.float32),
        pltpu.SemaphoreType.DMA,
        pltpu.SemaphoreType.DMA,
        pltpu.SemaphoreType.DMA,
        pltpu.SemaphoreType.DMA,
    ],
)
def _agg_kernel(u_hbm, srcp_hbm, dstp_hbm, cnt_hbm, agg_hbm,
                srcv, dstv, cnts, rows, accum, *sems):
    cid = lax.axis_index("c")
    sid = lax.axis_index("s")

    _zero_vmem_2d(rows.at[0], C, D)

    def zblk(k, carry):
        b = sid + k * NS

        @pl.when(b < NBLK2)
        def _():
            off = pl.multiple_of(b * C, 8)
            pltpu.sync_copy(rows.at[0], accum.at[pl.ds(off, C)])
        return carry

    lax.fori_loop(0, (NBLK2 + NS - 1) // NS, zblk, 0)
    plsc.subcore_barrier()

    def idx(ref, c):
        return ref.at[c]

    def gather(c, b):
        return pltpu.make_async_copy(u_hbm.at[idx(srcv, c)], rows.at[b],
                                     sems[b])

    def scat(c, b):
        gather(c, b).wait()
        pltpu.sync_copy(rows.at[b], accum.at[idx(dstv, c)], add=True)

    for q in range(2):
        p = sid * 2 + q
        pltpu.sync_copy(srcp_hbm.at[cid, p], srcv)
        pltpu.sync_copy(dstp_hbm.at[cid, p], dstv)
        pltpu.sync_copy(cnt_hbm.at[pl.ds(p * 16, 16)], cnts)
        cv = cnts[pl.ds(0, 16)]
        nch = jnp.where(cid == 0, cv[0], cv[1])

        for b in range(DEPTH):
            gather(b, b).start()

        def blk(j, carry):
            c0 = j * DEPTH
            for b in range(DEPTH):
                scat(c0 + b, b)
                gather(c0 + b + DEPTH, b).start()
            return carry

        lax.fori_loop(0, TARGET // DEPTH - 1, blk, 0)
        for b in range(DEPTH):
            scat(TARGET - DEPTH + b, b)

        @pl.when(nch > TARGET)
        def _():
            def rem(c, carry):
                gather(c, 0).start()
                scat(c, 0)
                return carry

            lax.fori_loop(TARGET, nch, rem, 0)
    plsc.subcore_barrier()

    def dblk(k, carry):
        b = sid + k * NS

        @pl.when(b < NBLK2)
        def _():
            off = pl.multiple_of(b * C, 8)
            pltpu.sync_copy(accum.at[pl.ds(off, C)],
                            agg_hbm.at[cid, pl.ds(off, C)])
        return carry

    lax.fori_loop(0, (NBLK2 + NS - 1) // NS, dblk, 0)


def _dinv_from_parts(degp):
    deg = degp[0, :N] + degp[1, :N] + 1.0
    return lax.rsqrt(jnp.maximum(deg, 1e-12))


def _first_tc(x_ref, w_ref, degp_ref, u_ref):
    dinv = _dinv_from_parts(degp_ref[...])
    h = jnp.dot(x_ref[...], w_ref[...], preferred_element_type=jnp.float32)
    u_ref[...] = h * dinv[:, None]


def _mid_tc(aggp_ref, u_ref, degp_ref, b_ref, g_ref, be_ref, w_ref, out_ref):
    dinv = _dinv_from_parts(degp_ref[...])
    agg = jnp.concatenate(
        [aggp_ref[0, :HALF], aggp_ref[1, :HALF]], axis=0) + u_ref[...]
    z = agg * dinv[:, None] + b_ref[...]
    mu = jnp.mean(z, axis=0)
    zc = z - mu
    var = jnp.mean(zc * zc, axis=0)
    h = g_ref[...] * zc * lax.rsqrt(var + 1e-5) + be_ref[...]
    h = jnp.maximum(h, 0.0)
    out_ref[...] = (
        jnp.dot(h, w_ref[...], preferred_element_type=jnp.float32)
        * dinv[:, None])


def _last_tc(aggp_ref, u_ref, degp_ref, b_ref, g_ref, be_ref, seg_ref,
             wout_ref, bout_ref, out_ref):
    dinv = _dinv_from_parts(degp_ref[...])
    agg = jnp.concatenate(
        [aggp_ref[0, :HALF], aggp_ref[1, :HALF]], axis=0) + u_ref[...]
    z = agg * dinv[:, None] + b_ref[...]
    mu = jnp.mean(z, axis=0)
    zc = z - mu
    var = jnp.mean(zc * zc, axis=0)
    h = g_ref[...] * zc * lax.rsqrt(var + 1e-5) + be_ref[...]

    seg = seg_ref[...]
    onehot = (seg[:, None] == lax.broadcasted_iota(jnp.int32, (1, G), 1)
              ).astype(jnp.float32)
    sums = lax.dot_general(onehot, h, (((0,), (0,)), ((), ())),
                           preferred_element_type=jnp.float32)
    cnt = jnp.sum(onehot, axis=0)
    p = sums / jnp.maximum(cnt, 1.0)[:, None]
    y = jnp.dot(p, wout_ref[...], preferred_element_type=jnp.float32)
    y = y + bout_ref[...]
    out_ref[...] = jnp.where(y > 0, y, 0.1 * y)


def _tc_call(body, out_shape, *args):
    return pl.pallas_call(
        body, out_shape=jax.ShapeDtypeStruct(out_shape, jnp.float32))(*args)


def kernel(x, edge_index, batch_seg, W1, b1, W2, b2, W3, b3,
           g1, be1, g2, be2, g3, be3, Wout, bout):
    src1d = edge_index[0]
    dst1d = edge_index[1]
    dst3d = edge_index[1].reshape(NW, NCHUNK, C)

    degp = _deg_kernel(dst3d)
    srcp, dstp, cntp = _part_kernel(src1d, dst1d)
    u1 = _tc_call(_first_tc, (N, D), x, W1, degp)
    agg1 = _agg_kernel(u1, srcp, dstp, cntp)
    u2 = _tc_call(_mid_tc, (N, D), agg1, u1, degp, b1, g1, be1, W2)
    agg2 = _agg_kernel(u2, srcp, dstp, cntp)
    u3 = _tc_call(_mid_tc, (N, D), agg2, u2, degp, b2, g2, be2, W3)
    agg3 = _agg_kernel(u3, srcp, dstp, cntp)
    out = _tc_call(_last_tc, (G, T), agg3, u3, degp, b3, g3, be3,
                   batch_seg, Wout, bout)
    return out


# speedup vs baseline: 2.8437x; 2.8437x over previous
import functools

import jax
import jax.numpy as jnp
from jax import lax
from jax.experimental import pallas as pl
from jax.experimental.pallas import tpu as pltpu
from jax.experimental.pallas import tpu_sc as plsc

N = 10000
D = 128
E = 320000
G = 512
T = 100

NC = 2
NS = 16
NW = NC * NS
C = 80
EPT = E // NW
NCHUNK = EPT // C
NGRP = 5
GRP = NCHUNK // NGRP
NBLK = N // C
NDEG = 10240
DEG_PT = NDEG // NS

_mesh = plsc.VectorSubcoreMesh(core_axis_name="c", subcore_axis_name="s")


def _zero_vmem_1d(ref, n):
    z16 = jnp.zeros((16,), jnp.float32)

    def body(i, carry):
        ref[pl.ds(i * 16, 16)] = z16
        return carry

    lax.fori_loop(0, n // 16, body, 0)


def _zero_vmem_2d(ref, rows, cols):
    z16 = jnp.zeros((16,), jnp.float32)

    def body(i, carry):
        r = i // (cols // 16)
        c = lax.rem(i, cols // 16) * 16
        ref[r, pl.ds(c, 16)] = z16
        return carry

    lax.fori_loop(0, rows * (cols // 16), body, 0)


@functools.partial(
    pl.kernel,
    out_type=jax.ShapeDtypeStruct((NC, NDEG), jnp.float32),
    mesh=_mesh,
    scratch_types=[
        pltpu.VMEM((NCHUNK, C), jnp.int32),
        pltpu.VMEM((C,), jnp.float32),
        pltpu.VMEM((DEG_PT,), jnp.float32),
        pltpu.VMEM_SHARED((NDEG,), jnp.float32),
    ],
)
def _deg_kernel(dst_hbm, degp_hbm, idx_v, ones_v, zbuf, deg_sh):
    cid = lax.axis_index("c")
    sid = lax.axis_index("s")
    w = cid * NS + sid

    pltpu.sync_copy(dst_hbm.at[w], idx_v)

    one16 = jnp.ones((16,), jnp.float32)
    for i in range(C // 16):
        ones_v[pl.ds(i * 16, 16)] = one16
    _zero_vmem_1d(zbuf, DEG_PT)
    pltpu.sync_copy(zbuf, deg_sh.at[pl.ds(sid * DEG_PT, DEG_PT)])
    plsc.subcore_barrier()

    def step(c, carry):
        pltpu.sync_copy(ones_v, deg_sh.at[idx_v.at[c]], add=True)
        return carry

    lax.fori_loop(0, NCHUNK, step, 0)
    plsc.subcore_barrier()

    pltpu.sync_copy(deg_sh.at[pl.ds(sid * DEG_PT, DEG_PT)],
                    degp_hbm.at[cid, pl.ds(sid * DEG_PT, DEG_PT)])


DEPTH = 4
HALF = N // NC
TRASH = HALF
NCH = 128
TARGET = 68
AROWS = 5056
NBLK2 = (HALF + C - 1) // C


@functools.partial(
    pl.kernel,
    out_type=(
        jax.ShapeDtypeStruct((2, NW, NCH, C), jnp.int32),
        jax.ShapeDtypeStruct((2, NW, NCH, C), jnp.int32),
        jax.ShapeDtypeStruct((NW * 16,), jnp.int32),
    ),
    mesh=_mesh,
    compiler_params=pltpu.CompilerParams(needs_layout_passes=False),
    scratch_types=[
        pltpu.VMEM((EPT,), jnp.int32),
        pltpu.VMEM((EPT,), jnp.int32),
        pltpu.VMEM((NCH, C), jnp.int32),
        pltpu.VMEM((NCH, C), jnp.int32),
        pltpu.VMEM((NCH, C), jnp.int32),
        pltpu.VMEM((NCH, C), jnp.int32),
        pltpu.VMEM((16,), jnp.int32),
    ],
)
def _part_kernel(src_hbm, dst_hbm, srcp_hbm, dstp_hbm, cnt_hbm,
                 sin, din, src_a, dst_a, src_b, dst_b, cnts):
    cid = lax.axis_index("c")
    sid = lax.axis_index("s")
    p = cid * NS + sid
    pltpu.sync_copy(src_hbm.at[pl.ds(p * EPT, EPT)], sin)
    pltpu.sync_copy(dst_hbm.at[pl.ds(p * EPT, EPT)], din)

    lane = lax.broadcasted_iota(jnp.int32, (16,), 0)

    def div_c(x):
        return (x * 52429) >> 22

    def rowcol(pos, m):
        r0 = div_c(pos)
        r = jnp.where(m, r0, NCH - 1)
        c = jnp.where(m, pos - r0 * C, lane)
        return r, c

    def step(i, cnt):
        ca, cb = cnt
        s16 = sin[pl.ds(i * 16, 16)]
        d16 = din[pl.ds(i * 16, 16)]
        m_a = d16 < HALF
        m_b = jnp.logical_not(m_a)
        cs_a = plsc.cumsum(m_a.astype(jnp.int32))
        na = cs_a[15]
        ra, cla = rowcol(ca + cs_a - 1, m_a)
        rb, clb = rowcol(cb + (lane - cs_a), m_b)
        plsc.store_scatter(src_a, [ra, cla], s16)
        plsc.store_scatter(dst_a, [ra, cla], d16)
        plsc.store_scatter(src_b, [rb, clb], s16)
        plsc.store_scatter(dst_b, [rb, clb], d16 - HALF)
        return (ca + na, cb + (16 - na))

    ca, cb = lax.fori_loop(0, EPT // 16, step,
                           (jnp.int32(0), jnp.int32(0)))

    zsrc = jnp.zeros((16,), jnp.int32)
    nspare = AROWS - HALF

    def pad(ref_s, ref_d, cnt):
        nch = (cnt + C - 1) // C
        nch = jnp.maximum((nch + DEPTH - 1) // DEPTH * DEPTH, DEPTH)
        end = nch * C

        def fill(k, carry):
            off = cnt + k * 16

            @pl.when(off < end)
            def _():
                offs = jnp.minimum(off + lane, NCH * C - 1)
                r = div_c(offs)
                c = offs - r * C
                tdst = HALF + offs % nspare
                plsc.store_scatter(ref_s, [r, c], zsrc)
                plsc.store_scatter(ref_d, [r, c], tdst)
            return carry

        lax.fori_loop(0, DEPTH * C // 16 + 1, fill, 0)
        return nch

    nch_a = pad(src_a, dst_a, ca)
    nch_b = pad(src_b, dst_b, cb)

    cnts[...] = jnp.where(lane == 0, nch_a, jnp.where(lane == 1, nch_b, 0))

    pltpu.sync_copy(src_a, srcp_hbm.at[0, p])
    pltpu.sync_copy(dst_a, dstp_hbm.at[0, p])
    pltpu.sync_copy(src_b, srcp_hbm.at[1, p])
    pltpu.sync_copy(dst_b, dstp_hbm.at[1, p])
    pltpu.sync_copy(cnts, cnt_hbm.at[pl.ds(p * 16, 16)])


@functools.partial(
    pl.kernel,
    out_type=jax.ShapeDtypeStruct((NC, AROWS, D), jnp.float32),
    mesh=_mesh,
    scratch_types=[
        pltpu.VMEM((NCH, C), jnp.int32),
        pltpu.VMEM((NCH, C), jnp.int32),
        pltpu.VMEM((16,), jnp.int32),
        pltpu.VMEM((DEPTH, C, D), jnp.float32),
        pltpu.VMEM_SHARED((AROWS, D), jnp.float32),
        pltpu.SemaphoreType.DMA,
        pltpu.SemaphoreType.DMA,
        pltpu.SemaphoreType.DMA,
        pltpu.SemaphoreType.DMA,
    ],
)
def _agg_kernel(u_hbm, srcp_hbm, dstp_hbm, cnt_hbm, agg_hbm,
                srcv, dstv, cnts, rows, accum, *sems):
    cid = lax.axis_index("c")
    sid = lax.axis_index("s")

    _zero_vmem_2d(rows.at[0], C, D)

    def zblk(k, carry):
        b = sid + k * NS

        @pl.when(b < NBLK2)
        def _():
            off = pl.multiple_of(b * C, 8)
            pltpu.sync_copy(rows.at[0], accum.at[pl.ds(off, C)])
        return carry

    lax.fori_loop(0, (NBLK2 + NS - 1) // NS, zblk, 0)
    plsc.subcore_barrier()

    def idx(ref, c):
        return ref.at[c]

    def gather(c, b):
        return pltpu.make_async_copy(u_hbm.at[idx(srcv, c)], rows.at[b],
                                     sems[b])

    def scat(c, b):
        gather(c, b).wait()
        pltpu.sync_copy(rows.at[b], accum.at[idx(dstv, c)], add=True)

    for q in range(2):
        p = sid * 2 + q
        pltpu.sync_copy(srcp_hbm.at[cid, p], srcv)
        pltpu.sync_copy(dstp_hbm.at[cid, p], dstv)
        pltpu.sync_copy(cnt_hbm.at[pl.ds(p * 16, 16)], cnts)
        cv = cnts[pl.ds(0, 16)]
        nch = jnp.where(cid == 0, cv[0], cv[1])

        for b in range(DEPTH):
            gather(b, b).start()

        def blk(j, carry):
            c0 = j * DEPTH
            for b in range(DEPTH):
                scat(c0 + b, b)
                gather(c0 + b + DEPTH, b).start()
            return carry

        lax.fori_loop(0, nch // DEPTH - 1, blk, 0)
        c0f = nch - DEPTH
        for b in range(DEPTH):
            scat(c0f + b, b)
    plsc.subcore_barrier()

    def dblk(k, carry):
        b = sid + k * NS

        @pl.when(b < NBLK2)
        def _():
            off = pl.multiple_of(b * C, 8)
            pltpu.sync_copy(accum.at[pl.ds(off, C)],
                            agg_hbm.at[cid, pl.ds(off, C)])
        return carry

    lax.fori_loop(0, (NBLK2 + NS - 1) // NS, dblk, 0)


def _dinv_from_parts(degp):
    deg = degp[0, :N] + degp[1, :N] + 1.0
    return lax.rsqrt(jnp.maximum(deg, 1e-12))


def _first_tc(x_ref, w_ref, degp_ref, u_ref):
    dinv = _dinv_from_parts(degp_ref[...])
    h = jnp.dot(x_ref[...], w_ref[...], preferred_element_type=jnp.float32)
    u_ref[...] = h * dinv[:, None]


def _mid_tc(aggp_ref, u_ref, degp_ref, b_ref, g_ref, be_ref, w_ref, out_ref):
    dinv = _dinv_from_parts(degp_ref[...])
    agg = jnp.concatenate(
        [aggp_ref[0, :HALF], aggp_ref[1, :HALF]], axis=0) + u_ref[...]
    z = agg * dinv[:, None] + b_ref[...]
    mu = jnp.mean(z, axis=0)
    zc = z - mu
    var = jnp.mean(zc * zc, axis=0)
    h = g_ref[...] * zc * lax.rsqrt(var + 1e-5) + be_ref[...]
    h = jnp.maximum(h, 0.0)
    out_ref[...] = (
        jnp.dot(h, w_ref[...], preferred_element_type=jnp.float32)
        * dinv[:, None])


def _last_tc(aggp_ref, u_ref, degp_ref, b_ref, g_ref, be_ref, seg_ref,
             wout_ref, bout_ref, out_ref):
    dinv = _dinv_from_parts(degp_ref[...])
    agg = jnp.concatenate(
        [aggp_ref[0, :HALF], aggp_ref[1, :HALF]], axis=0) + u_ref[...]
    z = agg * dinv[:, None] + b_ref[...]
    mu = jnp.mean(z, axis=0)
    zc = z - mu
    var = jnp.mean(zc * zc, axis=0)
    h = g_ref[...] * zc * lax.rsqrt(var + 1e-5) + be_ref[...]

    seg = seg_ref[...]
    onehot = (seg[:, None] == lax.broadcasted_iota(jnp.int32, (1, G), 1)
              ).astype(jnp.float32)
    sums = lax.dot_general(onehot, h, (((0,), (0,)), ((), ())),
                           preferred_element_type=jnp.float32)
    cnt = jnp.sum(onehot, axis=0)
    p = sums / jnp.maximum(cnt, 1.0)[:, None]
    y = jnp.dot(p, wout_ref[...], preferred_element_type=jnp.float32)
    y = y + bout_ref[...]
    out_ref[...] = jnp.where(y > 0, y, 0.1 * y)


def _tc_call(body, out_shape, *args):
    return pl.pallas_call(
        body, out_shape=jax.ShapeDtypeStruct(out_shape, jnp.float32))(*args)


def kernel(x, edge_index, batch_seg, W1, b1, W2, b2, W3, b3,
           g1, be1, g2, be2, g3, be3, Wout, bout):
    src1d = edge_index[0]
    dst1d = edge_index[1]
    dst3d = edge_index[1].reshape(NW, NCHUNK, C)

    degp = _deg_kernel(dst3d)
    srcp, dstp, cntp = _part_kernel(src1d, dst1d)
    u1 = _tc_call(_first_tc, (N, D), x, W1, degp)
    agg1 = _agg_kernel(u1, srcp, dstp, cntp)
    u2 = _tc_call(_mid_tc, (N, D), agg1, u1, degp, b1, g1, be1, W2)
    agg2 = _agg_kernel(u2, srcp, dstp, cntp)
    u3 = _tc_call(_mid_tc, (N, D), agg2, u2, degp, b2, g2, be2, W3)
    agg3 = _agg_kernel(u3, srcp, dstp, cntp)
    out = _tc_call(_last_tc, (G, T), agg3, u3, degp, b3, g3, be3,
                   batch_seg, Wout, bout)
    return out


# speedup vs baseline: 8.7060x; 3.0615x over previous
import functools

import jax
import jax.numpy as jnp
from jax import lax
from jax.experimental import pallas as pl
from jax.experimental.pallas import tpu as pltpu
from jax.experimental.pallas import tpu_sc as plsc

N = 10000
D = 128
E = 320000
G = 512
T = 100

NC = 2
NS = 16
NW = NC * NS
C = 80
EPT = E // NW
NCHUNK = EPT // C
NGRP = 5
GRP = NCHUNK // NGRP
NBLK = N // C
NDEG = 10240
DEG_PT = NDEG // NS

_mesh = plsc.VectorSubcoreMesh(core_axis_name="c", subcore_axis_name="s")


def _zero_vmem_1d(ref, n):
    z16 = jnp.zeros((16,), jnp.float32)

    def body(i, carry):
        ref[pl.ds(i * 16, 16)] = z16
        return carry

    lax.fori_loop(0, n // 16, body, 0)


def _zero_vmem_2d(ref, rows, cols):
    z16 = jnp.zeros((16,), jnp.float32)

    def body(i, carry):
        r = i // (cols // 16)
        c = lax.rem(i, cols // 16) * 16
        ref[r, pl.ds(c, 16)] = z16
        return carry

    lax.fori_loop(0, rows * (cols // 16), body, 0)


@functools.partial(
    pl.kernel,
    out_type=jax.ShapeDtypeStruct((NC, NDEG), jnp.float32),
    mesh=_mesh,
    scratch_types=[
        pltpu.VMEM((NCHUNK, C), jnp.int32),
        pltpu.VMEM((C,), jnp.float32),
        pltpu.VMEM((DEG_PT,), jnp.float32),
        pltpu.VMEM_SHARED((NDEG,), jnp.float32),
    ],
)
def _deg_kernel(dst_hbm, degp_hbm, idx_v, ones_v, zbuf, deg_sh):
    cid = lax.axis_index("c")
    sid = lax.axis_index("s")
    w = cid * NS + sid

    pltpu.sync_copy(dst_hbm.at[w], idx_v)

    one16 = jnp.ones((16,), jnp.float32)
    for i in range(C // 16):
        ones_v[pl.ds(i * 16, 16)] = one16
    _zero_vmem_1d(zbuf, DEG_PT)
    pltpu.sync_copy(zbuf, deg_sh.at[pl.ds(sid * DEG_PT, DEG_PT)])
    plsc.subcore_barrier()

    def step(c, carry):
        pltpu.sync_copy(ones_v, deg_sh.at[idx_v.at[c]], add=True)
        return carry

    lax.fori_loop(0, NCHUNK, step, 0)
    plsc.subcore_barrier()

    pltpu.sync_copy(deg_sh.at[pl.ds(sid * DEG_PT, DEG_PT)],
                    degp_hbm.at[cid, pl.ds(sid * DEG_PT, DEG_PT)])


@functools.partial(
    pl.kernel,
    out_type=jax.ShapeDtypeStruct((NC, N, D), jnp.float32),
    mesh=_mesh,
    scratch_types=[
        pltpu.VMEM((GRP, C), jnp.int32),
        pltpu.VMEM((GRP, C), jnp.int32),
        pltpu.VMEM((4, C, D), jnp.float32),
        pltpu.VMEM_SHARED((N, D), jnp.float32),
        pltpu.SemaphoreType.DMA,
        pltpu.SemaphoreType.DMA,
        pltpu.SemaphoreType.DMA,
        pltpu.SemaphoreType.DMA,
    ],
)
def _agg_kernel(u_hbm, src_hbm, dst_hbm, agg_hbm,
                srcv, dstv, rows, accum, *sems):
    cid = lax.axis_index("c")
    sid = lax.axis_index("s")
    w = cid * NS + sid

    _zero_vmem_2d(rows.at[0], C, D)

    def zblk(k, carry):
        b = sid + k * NS

        @pl.when(b < NBLK)
        def _():
            off = pl.multiple_of(b * C, 8)
            pltpu.sync_copy(rows.at[0], accum.at[pl.ds(off, C)])
        return carry

    lax.fori_loop(0, (NBLK + NS - 1) // NS, zblk, 0)
    plsc.subcore_barrier()

    def gather(c, b):
        return pltpu.make_async_copy(u_hbm.at[srcv.at[c]], rows.at[b], sems[b])

    def scat(c, b):
        gather(c, b).wait()
        pltpu.sync_copy(rows.at[b], accum.at[dstv.at[c]], add=True)

    for g in range(NGRP):
        pltpu.sync_copy(src_hbm.at[w, g], srcv)
        pltpu.sync_copy(dst_hbm.at[w, g], dstv)
        for b in range(4):
            gather(b, b).start()

        def blk(j, carry):
            c0 = j * 8
            for k in range(8):
                scat(c0 + k, k % 4)
                gather(c0 + k + 4, k % 4).start()
            return carry

        lax.fori_loop(0, 2, blk, 0)
        for c in range(16, GRP):
            scat(c, c % 4)
            if c + 4 < GRP:
                gather(c + 4, (c + 4) % 4).start()
    plsc.subcore_barrier()

    def dblk(k, carry):
        b = sid + k * NS

        @pl.when(b < NBLK)
        def _():
            off = pl.multiple_of(b * C, 8)
            pltpu.sync_copy(accum.at[pl.ds(off, C)],
                            agg_hbm.at[cid, pl.ds(off, C)])
        return carry

    lax.fori_loop(0, (NBLK + NS - 1) // NS, dblk, 0)


def _dinv_from_parts(degp):
    deg = degp[0, :N] + degp[1, :N] + 1.0
    return lax.rsqrt(jnp.maximum(deg, 1e-12))


def _first_tc(x_ref, w_ref, degp_ref, u_ref):
    dinv = _dinv_from_parts(degp_ref[...])
    h = jnp.dot(x_ref[...], w_ref[...], preferred_element_type=jnp.float32)
    u_ref[...] = h * dinv[:, None]


def _mid_tc(aggp_ref, u_ref, degp_ref, b_ref, g_ref, be_ref, w_ref, out_ref):
    dinv = _dinv_from_parts(degp_ref[...])
    agg = aggp_ref[0] + aggp_ref[1] + u_ref[...]
    z = agg * dinv[:, None] + b_ref[...]
    mu = jnp.mean(z, axis=0)
    zc = z - mu
    var = jnp.mean(zc * zc, axis=0)
    h = g_ref[...] * zc * lax.rsqrt(var + 1e-5) + be_ref[...]
    h = jnp.maximum(h, 0.0)
    out_ref[...] = (
        jnp.dot(h, w_ref[...], preferred_element_type=jnp.float32)
        * dinv[:, None])


def _last_tc(aggp_ref, u_ref, degp_ref, b_ref, g_ref, be_ref, seg_ref,
             wout_ref, bout_ref, out_ref):
    dinv = _dinv_from_parts(degp_ref[...])
    agg = aggp_ref[0] + aggp_ref[1] + u_ref[...]
    z = agg * dinv[:, None] + b_ref[...]
    mu = jnp.mean(z, axis=0)
    zc = z - mu
    var = jnp.mean(zc * zc, axis=0)
    h = g_ref[...] * zc * lax.rsqrt(var + 1e-5) + be_ref[...]

    seg = seg_ref[...]
    onehot = (seg[:, None] == lax.broadcasted_iota(jnp.int32, (1, G), 1)
              ).astype(jnp.float32)
    sums = lax.dot_general(onehot, h, (((0,), (0,)), ((), ())),
                           preferred_element_type=jnp.float32)
    cnt = jnp.sum(onehot, axis=0)
    p = sums / jnp.maximum(cnt, 1.0)[:, None]
    y = jnp.dot(p, wout_ref[...], preferred_element_type=jnp.float32)
    y = y + bout_ref[...]
    out_ref[...] = jnp.where(y > 0, y, 0.1 * y)


def _tc_call(body, out_shape, *args):
    return pl.pallas_call(
        body, out_shape=jax.ShapeDtypeStruct(out_shape, jnp.float32))(*args)


def kernel(x, edge_index, batch_seg, W1, b1, W2, b2, W3, b3,
           g1, be1, g2, be2, g3, be3, Wout, bout):
    src4d = edge_index[0].reshape(NW, NGRP, GRP, C)
    dst4d = edge_index[1].reshape(NW, NGRP, GRP, C)
    dst3d = edge_index[1].reshape(NW, NCHUNK, C)

    degp = _deg_kernel(dst3d)
    u1 = _tc_call(_first_tc, (N, D), x, W1, degp)
    agg1 = _agg_kernel(u1, src4d, dst4d)
    u2 = _tc_call(_mid_tc, (N, D), agg1, u1, degp, b1, g1, be1, W2)
    agg2 = _agg_kernel(u2, src4d, dst4d)
    u3 = _tc_call(_mid_tc, (N, D), agg2, u2, degp, b2, g2, be2, W3)
    agg3 = _agg_kernel(u3, src4d, dst4d)
    out = _tc_call(_last_tc, (G, T), agg3, u3, degp, b3, g3, be3,
                   batch_seg, Wout, bout)
    return out
